# Initial kernel scaffold; baseline (speedup 1.0000x reference)
import functools

import jax
import jax.numpy as jnp
from jax import lax
from jax.experimental import pallas as pl
from jax.experimental.pallas import tpu as pltpu
from jax.experimental.pallas import tpu_sc as plsc

D = 128
LANES = 16
EDGE_BATCH = 128
NC, NS = 2, 16
NW = NC * NS
PAD_EDGES = 256


def _build_seg_sum(n_seg: int, rounds: int):
    seg_per_tec = n_seg // (NW * rounds)
    seg_pad = seg_per_tec + 8
    mesh = plsc.VectorSubcoreMesh(core_axis_name="c", subcore_axis_name="s")

    @functools.partial(
        pl.kernel,
        out_type=(
            jax.ShapeDtypeStruct((n_seg, D), jnp.float32),
            jax.ShapeDtypeStruct((n_seg, LANES), jnp.float32),
        ),
        mesh=mesh,
        scratch_types=[
            pltpu.VMEM((seg_pad, D), jnp.float32),
            pltpu.VMEM((seg_pad, LANES), jnp.float32),
            pltpu.VMEM((EDGE_BATCH, D), jnp.float32),
            pltpu.VMEM((EDGE_BATCH,), jnp.int32),
            pltpu.VMEM((EDGE_BATCH,), jnp.int32),
            pltpu.VMEM((EDGE_BATCH, LANES), jnp.float32),
            pltpu.VMEM((16,), jnp.int32),
            pltpu.SemaphoreType.DMA,
        ],
    )
    def body(table, src, dst, offs, sum_out, cnt_out,
             acc, cnt, rows, idxb, dstb, ones, offw, sem):
        wid = lax.axis_index("s") * NC + lax.axis_index("c")
        one_v = jnp.full((LANES,), 1.0, jnp.float32)
        zero_v = jnp.zeros((LANES,), jnp.float32)

        @pl.loop(0, EDGE_BATCH)
        def _(i):
            ones[i, :] = one_v

        iota = lax.iota(jnp.int32, LANES)
        for r in range(rounds):
            j = r * NW + wid
            base = j * seg_per_tec
            w0 = lax.bitwise_and(j, -8)
            pltpu.sync_copy(offs.at[pl.ds(w0, 16)], offw)
            ov = offw[...]
            lane = j - w0
            lo = jnp.max(jnp.where(iota == lane, ov, -1))
            hi = jnp.max(jnp.where(iota == lane + 1, ov, -1))
            start = lax.bitwise_and(lo, -8)
            nb = lax.shift_right_logical(
                hi - start + (EDGE_BATCH - 1), EDGE_BATCH.bit_length() - 1)

            @pl.loop(0, seg_pad)
            def _(i):
                cnt[i, :] = zero_v
                for k in range(D // LANES):
                    acc[i, pl.ds(k * LANES, LANES)] = zero_v

            @pl.loop(0, nb)
            def _(bi):
                e = start + bi * EDGE_BATCH
                pltpu.sync_copy(src.at[pl.ds(e, EDGE_BATCH)], idxb)
                pltpu.sync_copy(dst.at[pl.ds(e, EDGE_BATCH)], dstb)
                for s in range(EDGE_BATCH // LANES):
                    sl = pl.ds(s * LANES, LANES)
                    dv = dstb[sl] - base
                    ok = (dv >= 0) & (dv < seg_per_tec)
                    dstb[sl] = jnp.where(ok, dv, seg_per_tec)
                pltpu.async_copy(table.at[idxb], rows, sem).wait()
                pltpu.sync_copy(rows, acc.at[dstb], add=True)
                pltpu.sync_copy(ones, cnt.at[dstb], add=True)

            pltpu.sync_copy(acc.at[pl.ds(0, seg_per_tec)],
                            sum_out.at[pl.ds(base, seg_per_tec)])
            pltpu.sync_copy(cnt.at[pl.ds(0, seg_per_tec)],
                            cnt_out.at[pl.ds(base, seg_per_tec)])

    return body


def _dense_layer(sum_rows, cnt_rows, x_dst, w_neigh, w_root, b, relu: bool):
    n = sum_rows.shape[0]
    blk = 512
    grid = n // blk
    cnt_rs = cnt_rows.reshape(n // 8, 128)

    def body(s_ref, c_ref, xd_ref, wn_ref, wr_ref, b_ref, o_ref):
        s = s_ref[...]
        c = c_ref[...].reshape(blk, LANES).max(axis=1)
        mean = s * (1.0 / jnp.clip(c, 1.0, None))[:, None]
        dn = (((1,), (1,)), ((), ()))
        o = (lax.dot_general(mean, wn_ref[...], dn,
                             preferred_element_type=jnp.float32)
             + lax.dot_general(xd_ref[...], wr_ref[...], dn,
                               preferred_element_type=jnp.float32)
             + b_ref[...])
        if relu:
            o = jnp.maximum(o, 0.0)
        o_ref[...] = o

    return pl.pallas_call(
        body,
        grid=(grid,),
        in_specs=[
            pl.BlockSpec((blk, D), lambda i: (i, 0)),
            pl.BlockSpec((blk * LANES // 128, 128), lambda i: (i, 0)),
            pl.BlockSpec((blk, D), lambda i: (i, 0)),
            pl.BlockSpec((D, D), lambda i: (0, 0)),
            pl.BlockSpec((D, D), lambda i: (0, 0)),
            pl.BlockSpec((1, D), lambda i: (0, 0)),
        ],
        out_specs=pl.BlockSpec((blk, D), lambda i: (i, 0)),
        out_shape=jax.ShapeDtypeStruct((n, D), jnp.float32),
    )(sum_rows, cnt_rs, x_dst, w_neigh, w_root, b.reshape(1, D))


def _layer(x_src, x_dst, src, dst, n_seg, rounds, w_neigh, w_root, b, relu):
    e = src.shape[0]
    seg_per_tec = n_seg // (NW * rounds)
    n_bound = NW * rounds + 1
    bounds = jnp.arange(n_bound, dtype=jnp.int32) * seg_per_tec
    offs = jnp.searchsorted(dst, bounds).astype(jnp.int32)
    off_pad = -(-(n_bound + 8) // 8) * 8
    offs = jnp.pad(offs, (0, off_pad - n_bound), constant_values=e)
    src_p = jnp.pad(src.astype(jnp.int32), (0, PAD_EDGES))
    dst_p = jnp.pad(dst.astype(jnp.int32), (0, PAD_EDGES),
                    constant_values=jnp.int32(1 << 24))
    seg_sum = _build_seg_sum(n_seg, rounds)
    ssum, cnt = seg_sum(x_src, src_p, dst_p, offs)
    return _dense_layer(ssum, cnt, x_dst, w_neigh, w_root, b, relu)


def kernel(x, src_l0, dst_l0, src_l1, dst_l1, n_target_l0, n_target_l1,
           W0_neigh, W0_root, b0, W1_neigh, W1_root, b1):
    N1, N2 = 32768, 2048
    d0 = (dst_l0 + (n_target_l0 - N1)).astype(jnp.int32)
    d1 = (dst_l1 + (n_target_l1 - N2)).astype(jnp.int32)
    h = _layer(x, x, src_l0.astype(jnp.int32), d0, N1, 2,
               W0_neigh, W0_root, b0, relu=True)
    out = _layer(h, h, src_l1.astype(jnp.int32), d1, N2, 1,
                 W1_neigh, W1_root, b1, relu=False)
    return out


# speedup vs baseline: 4.5287x; 4.5287x over previous
import functools

import jax
import jax.numpy as jnp
from jax import lax
from jax.experimental import pallas as pl
from jax.experimental.pallas import tpu as pltpu
from jax.experimental.pallas import tpu_sc as plsc

D = 128
LANES = 16
EDGE_BATCH = 128
NC, NS = 2, 16
NW = NC * NS
PAD_EDGES = 256


def _build_seg_mean(n_seg: int, rounds: int):
    seg_per_tec = n_seg // (NW * rounds)
    seg_pad = seg_per_tec + 8
    cnt_pad = seg_per_tec + 16
    chunk = min(128, seg_per_tec)
    mesh = plsc.VectorSubcoreMesh(core_axis_name="c", subcore_axis_name="s")

    @functools.partial(
        pl.kernel,
        out_type=jax.ShapeDtypeStruct((n_seg, D), jnp.float32),
        mesh=mesh,
        compiler_params=pltpu.CompilerParams(needs_layout_passes=False),
        scratch_types=[
            pltpu.VMEM_SHARED((NS * seg_pad, D), jnp.float32),
            pltpu.VMEM((cnt_pad,), jnp.float32),
            pltpu.VMEM((EDGE_BATCH, D), jnp.float32),
            pltpu.VMEM((128, D), jnp.float32),
            pltpu.VMEM((EDGE_BATCH,), jnp.int32),
            pltpu.VMEM((EDGE_BATCH,), jnp.int32),
            pltpu.VMEM((16,), jnp.int32),
            pltpu.SemaphoreType.DMA,
        ],
    )
    def body(table, src, dst, offs, mean_out,
             acc, cnt, rows, zbuf, idxb, dstb, offw, sem):
        cid = lax.axis_index("c")
        sid = lax.axis_index("s")
        wid = sid * NC + cid
        slot = pl.multiple_of(sid * seg_pad, 8)
        one_v = jnp.full((LANES,), 1.0, jnp.float32)
        zero_v = jnp.zeros((LANES,), jnp.float32)
        iota = lax.iota(jnp.int32, LANES)

        @pl.loop(0, 128)
        def _(i):
            for k in range(D // LANES):
                zbuf[i, pl.ds(k * LANES, LANES)] = zero_v

        for r in range(rounds):
            j = r * NW + wid
            base = pl.multiple_of(j * seg_per_tec, 8)
            w0 = pl.multiple_of(lax.bitwise_and(j, -8), 8)
            pltpu.sync_copy(offs.at[pl.ds(w0, 16)], offw)
            ov = offw[...]
            lane = j - w0
            lo = jnp.max(jnp.where(iota == lane, ov, -1))
            hi = jnp.max(jnp.where(iota == lane + 1, ov, -1))
            start = pl.multiple_of(lax.bitwise_and(lo, -8), 8)
            nb = lax.shift_right_logical(
                hi - start + (EDGE_BATCH - 1), EDGE_BATCH.bit_length() - 1)

            off = 0
            while off < seg_pad:
                c = min(128, seg_pad - off)
                pltpu.sync_copy(zbuf.at[pl.ds(0, c)],
                                acc.at[pl.ds(slot + off, c)])
                off += c
            for i in range(cnt_pad // LANES):
                cnt[pl.ds(i * LANES, LANES)] = zero_v

            @pl.loop(0, nb)
            def _(bi):
                e = pl.multiple_of(start + bi * EDGE_BATCH, 8)
                pltpu.sync_copy(src.at[pl.ds(e, EDGE_BATCH)], idxb)
                pltpu.sync_copy(dst.at[pl.ds(e, EDGE_BATCH)], dstb)
                for s in range(EDGE_BATCH // LANES):
                    sl = pl.ds(s * LANES, LANES)
                    dv = dstb[sl] - base
                    ok = (dv >= 0) & (dv < seg_per_tec)
                    dvc = jnp.where(ok, dv, seg_per_tec)
                    plsc.addupdate_scatter(cnt, [dvc], one_v)
                    dstb[sl] = dvc + slot
                pltpu.async_copy(table.at[idxb], rows, sem).wait()
                pltpu.sync_copy(rows, acc.at[dstb], add=True)

            for c in range(seg_per_tec // chunk):
                pltpu.sync_copy(acc.at[pl.ds(slot + c * chunk, chunk)],
                                rows.at[pl.ds(0, chunk)])
                for g in range(chunk // LANES):
                    cv = cnt[pl.ds(c * chunk + g * LANES, LANES)]
                    inv16 = 1.0 / jnp.maximum(cv, 1.0)

                    @pl.loop(0, LANES)
                    def _(i):
                        sc = jnp.max(jnp.where(iota == i, inv16, -1.0))
                        row = g * LANES + i
                        for k in range(D // LANES):
                            sl = pl.ds(k * LANES, LANES)
                            rows[row, sl] = rows[row, sl] * sc

                pltpu.sync_copy(rows.at[pl.ds(0, chunk)],
                                mean_out.at[pl.ds(base + c * chunk, chunk)])

    return body


def _dense_layer(mean_rows, x_dst, w_neigh, w_root, b, relu: bool):
    n = mean_rows.shape[0]
    blk = 512
    grid = n // blk

    def body(m_ref, xd_ref, wn_ref, wr_ref, b_ref, o_ref):
        dn = (((1,), (1,)), ((), ()))
        o = (lax.dot_general(m_ref[...], wn_ref[...], dn,
                             preferred_element_type=jnp.float32)
             + lax.dot_general(xd_ref[...], wr_ref[...], dn,
                               preferred_element_type=jnp.float32)
             + b_ref[...])
        if relu:
            o = jnp.maximum(o, 0.0)
        o_ref[...] = o

    return pl.pallas_call(
        body,
        grid=(grid,),
        in_specs=[
            pl.BlockSpec((blk, D), lambda i: (i, 0)),
            pl.BlockSpec((blk, D), lambda i: (i, 0)),
            pl.BlockSpec((D, D), lambda i: (0, 0)),
            pl.BlockSpec((D, D), lambda i: (0, 0)),
            pl.BlockSpec((1, D), lambda i: (0, 0)),
        ],
        out_specs=pl.BlockSpec((blk, D), lambda i: (i, 0)),
        out_shape=jax.ShapeDtypeStruct((n, D), jnp.float32),
    )(mean_rows, x_dst, w_neigh, w_root, b.reshape(1, D))


def _layer(x_src, x_dst, src, dst, n_seg, rounds, w_neigh, w_root, b, relu):
    e = src.shape[0]
    seg_per_tec = n_seg // (NW * rounds)
    n_bound = NW * rounds + 1
    bounds = jnp.arange(n_bound, dtype=jnp.int32) * seg_per_tec
    offs = jnp.searchsorted(dst, bounds).astype(jnp.int32)
    off_pad = -(-(n_bound + 8) // 8) * 8
    offs = jnp.pad(offs, (0, off_pad - n_bound), constant_values=e)
    src_p = jnp.pad(src.astype(jnp.int32), (0, PAD_EDGES))
    dst_p = jnp.pad(dst.astype(jnp.int32), (0, PAD_EDGES),
                    constant_values=jnp.int32(1 << 24))
    seg_mean = _build_seg_mean(n_seg, rounds)
    mean = seg_mean(x_src, src_p, dst_p, offs)
    return _dense_layer(mean, x_dst, w_neigh, w_root, b, relu)


def kernel(x, src_l0, dst_l0, src_l1, dst_l1, n_target_l0, n_target_l1,
           W0_neigh, W0_root, b0, W1_neigh, W1_root, b1):
    N1, N2 = 32768, 2048
    d0 = (dst_l0 + (n_target_l0 - N1)).astype(jnp.int32)
    d1 = (dst_l1 + (n_target_l1 - N2)).astype(jnp.int32)
    h = _layer(x, x, src_l0.astype(jnp.int32), d0, N1, 4,
               W0_neigh, W0_root, b0, relu=True)
    out = _layer(h, h, src_l1.astype(jnp.int32), d1, N2, 1,
                 W1_neigh, W1_root, b1, relu=False)
    return out


# speedup vs baseline: 6.3019x; 1.3916x over previous
import functools

import jax
import jax.numpy as jnp
from jax import lax
from jax.experimental import pallas as pl
from jax.experimental.pallas import tpu as pltpu
from jax.experimental.pallas import tpu_sc as plsc

D = 128
LANES = 16
EDGE_BATCH = 128
NC, NS = 2, 16
NW = NC * NS
PAD_EDGES = 768


def _build_seg_mean(n_seg: int, rounds: int):
    seg_per_tec = n_seg // (NW * rounds)
    seg_pad = seg_per_tec + 8
    cnt_pad = seg_per_tec + 16
    chunk = min(128, seg_per_tec)
    mesh = plsc.VectorSubcoreMesh(core_axis_name="c", subcore_axis_name="s")

    @functools.partial(
        pl.kernel,
        out_type=jax.ShapeDtypeStruct((n_seg, D), jnp.float32),
        mesh=mesh,
        compiler_params=pltpu.CompilerParams(needs_layout_passes=False),
        scratch_types=[
            pltpu.VMEM_SHARED((NS * seg_pad, D), jnp.float32),
            pltpu.VMEM((cnt_pad,), jnp.float32),
            pltpu.VMEM((EDGE_BATCH, D), jnp.float32),
            pltpu.VMEM((EDGE_BATCH, D), jnp.float32),
            pltpu.VMEM((128, D), jnp.float32),
            pltpu.VMEM((EDGE_BATCH,), jnp.int32),
            pltpu.VMEM((EDGE_BATCH,), jnp.int32),
            pltpu.VMEM((EDGE_BATCH,), jnp.int32),
            pltpu.VMEM((EDGE_BATCH,), jnp.int32),
            pltpu.VMEM((EDGE_BATCH,), jnp.int32),
            pltpu.VMEM((EDGE_BATCH,), jnp.int32),
            pltpu.VMEM((16,), jnp.int32),
            pltpu.SemaphoreType.DMA,
            pltpu.SemaphoreType.DMA,
            pltpu.SemaphoreType.DMA,
            pltpu.SemaphoreType.DMA,
            pltpu.SemaphoreType.DMA,
            pltpu.SemaphoreType.DMA,
        ],
    )
    def body(table, src, dst, offs, mean_out,
             acc, cnt, rows0, rows1, zbuf, idxb0, idxb1, dstr0, dstr1,
             dsts0, dsts1, offw, semA0, semA1, semG0, semG1, semS0, semS1):
        rows_ = (rows0, rows1)
        idxb_ = (idxb0, idxb1)
        dstr_ = (dstr0, dstr1)
        dsts_ = (dsts0, dsts1)
        semA_ = (semA0, semA1)
        semG_ = (semG0, semG1)
        semS_ = (semS0, semS1)
        cid = lax.axis_index("c")
        sid = lax.axis_index("s")
        wid = sid * NC + cid
        slot = pl.multiple_of(sid * seg_pad, 8)
        one_v = jnp.full((LANES,), 1.0, jnp.float32)
        zero_v = jnp.zeros((LANES,), jnp.float32)
        iota = lax.iota(jnp.int32, LANES)

        @pl.loop(0, 128)
        def _(i):
            for k in range(D // LANES):
                zbuf[i, pl.ds(k * LANES, LANES)] = zero_v

        for r in range(rounds):
            j = r * NW + wid
            base = pl.multiple_of(j * seg_per_tec, 8)
            w0 = pl.multiple_of(lax.bitwise_and(j, -8), 8)
            pltpu.sync_copy(offs.at[pl.ds(w0, 16)], offw)
            ov = offw[...]
            lane = j - w0
            lo = jnp.max(jnp.where(iota == lane, ov, -1))
            hi = jnp.max(jnp.where(iota == lane + 1, ov, -1))
            start = pl.multiple_of(lax.bitwise_and(lo, -8), 8)
            nb = lax.shift_right_logical(
                hi - start + (EDGE_BATCH - 1), EDGE_BATCH.bit_length() - 1)

            off = 0
            while off < seg_pad:
                c = min(128, seg_pad - off)
                pltpu.sync_copy(zbuf.at[pl.ds(0, c)],
                                acc.at[pl.ds(slot + off, c)])
                off += c
            for i in range(cnt_pad // LANES):
                cnt[pl.ds(i * LANES, LANES)] = zero_v

            def issue_idxdst(e, p):
                pltpu.async_copy(src.at[pl.ds(e, EDGE_BATCH)],
                                 idxb_[p], semA_[p])
                pltpu.async_copy(dst.at[pl.ds(e, EDGE_BATCH)],
                                 dstr_[p], semA_[p])

            def wait_idxdst(p):
                pltpu.make_async_copy(src.at[pl.ds(0, EDGE_BATCH)],
                                      idxb_[p], semA_[p]).wait()
                pltpu.make_async_copy(dst.at[pl.ds(0, EDGE_BATCH)],
                                      dstr_[p], semA_[p]).wait()

            def compute(p):
                for s in range(EDGE_BATCH // LANES):
                    sl = pl.ds(s * LANES, LANES)
                    dv = dstr_[p][sl] - base
                    ok = (dv >= 0) & (dv < seg_per_tec)
                    dvc = jnp.where(ok, dv, seg_per_tec)
                    plsc.addupdate_scatter(cnt, [dvc], one_v)
                    dsts_[p][sl] = dvc + slot

            def wait_gather(p):
                pltpu.make_async_copy(table.at[idxb_[p]],
                                      rows_[p], semG_[p]).wait()

            def wait_scatter(p):
                pltpu.make_async_copy(rows_[p], acc.at[dsts_[p]],
                                      semS_[p]).wait()

            issue_idxdst(start, 0)
            issue_idxdst(pl.multiple_of(start + EDGE_BATCH, 8), 1)
            wait_idxdst(0)
            compute(0)
            pltpu.async_copy(table.at[idxb_[0]], rows_[0], semG_[0])

            npair = lax.shift_right_logical(nb + 2, 1)

            @pl.loop(0, npair)
            def _(g):
                for p in (0, 1):
                    q = 1 - p
                    i_val = g * 2 + p
                    wait_gather(p)
                    pltpu.async_copy(rows_[p], acc.at[dsts_[p]],
                                     semS_[p], add=True)
                    wait_idxdst(q)
                    if p == 1:
                        wait_scatter(q)
                    else:
                        @pl.when(g > 0)
                        def _():
                            wait_scatter(q)
                    compute(q)
                    pltpu.async_copy(table.at[idxb_[q]], rows_[q], semG_[q])
                    e2 = pl.multiple_of(
                        start + (i_val + 2) * EDGE_BATCH, 8)
                    issue_idxdst(e2, p)

            wait_scatter(1)
            wait_gather(0)
            wait_idxdst(1)

            for c in range(seg_per_tec // chunk):
                pltpu.sync_copy(acc.at[pl.ds(slot + c * chunk, chunk)],
                                rows0.at[pl.ds(0, chunk)])
                for g in range(chunk // LANES):
                    cv = cnt[pl.ds(c * chunk + g * LANES, LANES)]
                    inv16 = 1.0 / jnp.maximum(cv, 1.0)

                    @pl.loop(0, LANES)
                    def _(i):
                        sc = jnp.max(jnp.where(iota == i, inv16, -1.0))
                        row = g * LANES + i
                        for k in range(D // LANES):
                            sl = pl.ds(k * LANES, LANES)
                            rows0[row, sl] = rows0[row, sl] * sc

                pltpu.sync_copy(rows0.at[pl.ds(0, chunk)],
                                mean_out.at[pl.ds(base + c * chunk, chunk)])

    return body


def _dense_layer(mean_rows, x_dst, w_neigh, w_root, b, relu: bool):
    n = mean_rows.shape[0]
    blk = 512
    grid = n // blk

    def body(m_ref, xd_ref, wn_ref, wr_ref, b_ref, o_ref):
        dn = (((1,), (1,)), ((), ()))
        o = (lax.dot_general(m_ref[...], wn_ref[...], dn,
                             preferred_element_type=jnp.float32)
             + lax.dot_general(xd_ref[...], wr_ref[...], dn,
                               preferred_element_type=jnp.float32)
             + b_ref[...])
        if relu:
            o = jnp.maximum(o, 0.0)
        o_ref[...] = o

    return pl.pallas_call(
        body,
        grid=(grid,),
        in_specs=[
            pl.BlockSpec((blk, D), lambda i: (i, 0)),
            pl.BlockSpec((blk, D), lambda i: (i, 0)),
            pl.BlockSpec((D, D), lambda i: (0, 0)),
            pl.BlockSpec((D, D), lambda i: (0, 0)),
            pl.BlockSpec((1, D), lambda i: (0, 0)),
        ],
        out_specs=pl.BlockSpec((blk, D), lambda i: (i, 0)),
        out_shape=jax.ShapeDtypeStruct((n, D), jnp.float32),
    )(mean_rows, x_dst, w_neigh, w_root, b.reshape(1, D))


def _layer(x_src, x_dst, src, dst, n_seg, rounds, w_neigh, w_root, b, relu):
    e = src.shape[0]
    seg_per_tec = n_seg // (NW * rounds)
    n_bound = NW * rounds + 1
    bounds = jnp.arange(n_bound, dtype=jnp.int32) * seg_per_tec
    offs = jnp.searchsorted(dst, bounds).astype(jnp.int32)
    off_pad = -(-(n_bound + 8) // 8) * 8
    offs = jnp.pad(offs, (0, off_pad - n_bound), constant_values=e)
    src_p = jnp.pad(src.astype(jnp.int32), (0, PAD_EDGES))
    dst_p = jnp.pad(dst.astype(jnp.int32), (0, PAD_EDGES),
                    constant_values=jnp.int32(1 << 24))
    seg_mean = _build_seg_mean(n_seg, rounds)
    mean = seg_mean(x_src, src_p, dst_p, offs)
    return _dense_layer(mean, x_dst, w_neigh, w_root, b, relu)


def kernel(x, src_l0, dst_l0, src_l1, dst_l1, n_target_l0, n_target_l1,
           W0_neigh, W0_root, b0, W1_neigh, W1_root, b1):
    N1, N2 = 32768, 2048
    d0 = (dst_l0 + (n_target_l0 - N1)).astype(jnp.int32)
    d1 = (dst_l1 + (n_target_l1 - N2)).astype(jnp.int32)
    h = _layer(x, x, src_l0.astype(jnp.int32), d0, N1, 4,
               W0_neigh, W0_root, b0, relu=True)
    out = _layer(h, h, src_l1.astype(jnp.int32), d1, N2, 1,
                 W1_neigh, W1_root, b1, relu=False)
    return out
